# Initial kernel scaffold; baseline (speedup 1.0000x reference)
import jax
import jax.numpy as jnp
from jax.experimental import pallas as pl


def kernel(x, W1, b1, W2, b2, a_att, r_att, aux, edges):
    raise NotImplementedError("write your pallas kernel here")


# speedup vs baseline: 20.1804x; 20.1804x over previous
import functools

import jax
import jax.numpy as jnp
from jax import lax
from jax.experimental import pallas as pl
from jax.experimental.pallas import tpu as pltpu
from jax.experimental.pallas import tpu_sc as plsc

N = 10000
NP = 10240
NPH = NP // 2
E = 320000
F = 128
DA = 4
NCH = 2 * DA
NC, NS, L = 2, 16, 16
NW = NC * NS
EPW = E // NW
EPS = E // NS
RPT = NPH // NS

_mesh = plsc.VectorSubcoreMesh(
    core_axis_name="c", subcore_axis_name="s", num_cores=NC, num_subcores=NS)

_sc_params = pltpu.CompilerParams(needs_layout_passes=False)


def _wid():
    return lax.axis_index("s") * NC + lax.axis_index("c")


_CA = 2000


@functools.partial(
    pl.kernel,
    out_type=jax.ShapeDtypeStruct((NW * NCH * NP,), jnp.float32),
    mesh=_mesh,
    compiler_params=_sc_params,
    scratch_types=[
        pltpu.VMEM((NCH * NP,), jnp.float32),
        pltpu.VMEM((_CA,), jnp.int32),
        pltpu.VMEM((_CA,), jnp.int32),
        pltpu.VMEM((DA * _CA,), jnp.float32),
    ],
)
def _sc_rowsums(dst_h, src_h, valst_h, parts_h, acc, dbuf, sbuf, vbuf):
    wid = _wid()
    zf = jnp.zeros((L,), jnp.float32)

    def _z(j, _):
        acc[pl.ds(j * L, L)] = zf
        return _
    lax.fori_loop(0, NCH * NP // L, _z, None)

    def _chunk(g, _):
        base = wid * EPW + g * _CA
        pltpu.sync_copy(dst_h.at[pl.ds(base, _CA)], dbuf)
        pltpu.sync_copy(src_h.at[pl.ds(base, _CA)], sbuf)
        for i in range(DA):
            pltpu.sync_copy(valst_h.at[pl.ds(i * E + base, _CA)],
                            vbuf.at[pl.ds(i * _CA, _CA)])

        def _vec(j, _):
            dv = dbuf[pl.ds(j * L, L)]
            sv = sbuf[pl.ds(j * L, L)]
            for i in range(DA):
                v = vbuf[pl.ds(i * _CA + j * L, L)]
                if i == 0:
                    v = jnp.where(v == 0.0, 0.1, v)
                plsc.addupdate_scatter(acc, [dv + i * NP], v)
                plsc.addupdate_scatter(acc, [sv + (DA + i) * NP], v)
            return _
        lax.fori_loop(0, _CA // L, _vec, None)
        return _
    lax.fori_loop(0, EPW // _CA, _chunk, None)

    pltpu.sync_copy(acc, parts_h.at[pl.ds(wid * (NCH * NP), NCH * NP)])


def _tc_reduce_inv(parts, att_log):
    def body(p_ref, a_ref, rinv_ref, attw_ref):
        s = jnp.sum(p_ref[...], axis=0, keepdims=True)
        rinv_ref[...] = 1.0 / s
        a = a_ref[0:DA, :]
        r = a_ref[DA:NCH, :]
        ma = jnp.max(a, axis=0, keepdims=True)
        mr = jnp.max(r, axis=0, keepdims=True)
        ea = jnp.exp(a - ma)
        er = jnp.exp(r - mr)
        wa = ea / jnp.sum(ea, axis=0, keepdims=True)
        wr = er / jnp.sum(er, axis=0, keepdims=True)
        attw_ref[...] = jnp.concatenate([wa, wr], axis=0)

    return pl.pallas_call(
        body,
        out_shape=[jax.ShapeDtypeStruct((1, NCH * NP), jnp.float32),
                   jax.ShapeDtypeStruct((NCH, F), jnp.float32)],
    )(parts, att_log)


_CB = 2000


@functools.partial(
    pl.kernel,
    out_type=[jax.ShapeDtypeStruct((E,), jnp.float32),
              jax.ShapeDtypeStruct((E,), jnp.float32)],
    mesh=_mesh,
    compiler_params=_sc_params,
    scratch_types=[
        pltpu.VMEM((NCH * NP,), jnp.float32),
        pltpu.VMEM((NCH * F,), jnp.float32),
        pltpu.VMEM((_CB,), jnp.int32),
        pltpu.VMEM((_CB,), jnp.int32),
        pltpu.VMEM((DA * _CB,), jnp.float32),
        pltpu.VMEM((_CB,), jnp.float32),
        pltpu.VMEM((_CB,), jnp.float32),
    ],
)
def _sc_weights(rinv_h, attw_h, dst_h, src_h, valst_h, wa_h, wr_h,
                rinv, attw, dbuf, sbuf, vbuf, wabuf, wrbuf):
    wid = _wid()
    pltpu.sync_copy(rinv_h, rinv)
    pltpu.sync_copy(attw_h, attw)

    def _chunk(g, _):
        base = wid * EPW + g * _CB
        pltpu.sync_copy(dst_h.at[pl.ds(base, _CB)], dbuf)
        pltpu.sync_copy(src_h.at[pl.ds(base, _CB)], sbuf)
        for i in range(DA):
            pltpu.sync_copy(valst_h.at[pl.ds(i * E + base, _CB)],
                            vbuf.at[pl.ds(i * _CB, _CB)])

        def _vec(j, _):
            dv = dbuf[pl.ds(j * L, L)]
            sv = sbuf[pl.ds(j * L, L)]
            wa = jnp.zeros((L,), jnp.float32)
            wr = jnp.zeros((L,), jnp.float32)
            for i in range(DA):
                v = vbuf[pl.ds(i * _CB + j * L, L)]
                if i == 0:
                    v = jnp.where(v == 0.0, 0.1, v)
                ia = plsc.load_gather(rinv, [dv + i * NP])
                ir = plsc.load_gather(rinv, [sv + (DA + i) * NP])
                wa = wa + v * ia * attw[pl.ds(i * F, L)]
                wr = wr + v * ir * attw[pl.ds((DA + i) * F, L)]
            wabuf[pl.ds(j * L, L)] = wa
            wrbuf[pl.ds(j * L, L)] = wr
            return _
        lax.fori_loop(0, _CB // L, _vec, None)
        pltpu.sync_copy(wabuf, wa_h.at[pl.ds(base, _CB)])
        pltpu.sync_copy(wrbuf, wr_h.at[pl.ds(base, _CB)])
        return _
    lax.fori_loop(0, EPW // _CB, _chunk, None)


_CC = 400
_ZR = 64


@functools.partial(
    pl.kernel,
    out_type=jax.ShapeDtypeStruct((NP, F), jnp.float32),
    mesh=_mesh,
    compiler_params=_sc_params,
    scratch_types=[
        pltpu.VMEM((_ZR, F), jnp.float32),
        pltpu.VMEM((_CC, F), jnp.float32),
        pltpu.VMEM((_CC,), jnp.int32),
        pltpu.VMEM((_CC,), jnp.int32),
        pltpu.VMEM((_CC,), jnp.float32),
        pltpu.VMEM_SHARED((NPH + 8, F), jnp.float32),
        pltpu.SemaphoreType.DMA,
    ],
)
def _sc_spmm(sup_h, gidx_h, sidx_h, w_h, out_h,
             zbuf, rows, gbuf, sbuf, wbuf, acc_sh, sem):
    cid = lax.axis_index("c")
    sid = lax.axis_index("s")
    zf = jnp.zeros((L,), jnp.float32)

    def _z(r, _):
        for k in range(F // L):
            zbuf[r, pl.ds(k * L, L)] = zf
        return _
    lax.fori_loop(0, _ZR, _z, None)
    for j in range(RPT // _ZR):
        pltpu.sync_copy(zbuf, acc_sh.at[pl.ds(sid * RPT + j * _ZR, _ZR)])

    @pl.when(sid == 0)
    def _():
        pltpu.sync_copy(zbuf.at[pl.ds(0, 8)], acc_sh.at[pl.ds(NPH, 8)])

    plsc.subcore_barrier()
    lo = cid * NPH

    def _chunk(g, _):
        base = sid * EPS + g * _CC
        pltpu.sync_copy(gidx_h.at[pl.ds(base, _CC)], gbuf)
        pltpu.async_copy(sup_h.at[gbuf], rows, sem).wait()
        pltpu.sync_copy(w_h.at[pl.ds(base, _CC)], wbuf)
        pltpu.sync_copy(sidx_h.at[pl.ds(base, _CC)], sbuf)

        def _local(j, _):
            lv = sbuf[pl.ds(j * L, L)] - lo
            ok = (lv >= 0) & (lv < NPH)
            sbuf[pl.ds(j * L, L)] = jnp.where(ok, lv, NPH)
            return _
        lax.fori_loop(0, _CC // L, _local, None)

        def _scale(r, _):
            wv = plsc.load_gather(wbuf, [lax.broadcast_in_dim(r, (L,), ())])
            for k in range(F // L):
                rows[r, pl.ds(k * L, L)] = rows[r, pl.ds(k * L, L)] * wv
            return _
        lax.fori_loop(0, _CC, _scale, None)
        pltpu.sync_copy(rows, acc_sh.at[sbuf], add=True)
        return _
    lax.fori_loop(0, EPS // _CC, _chunk, None)

    plsc.subcore_barrier()
    pltpu.sync_copy(acc_sh.at[pl.ds(sid * RPT, RPT)],
                    out_h.at[pl.ds(cid * NPH + sid * RPT, RPT)])


_BR = 2000


def _tc_linear(x, W, b):
    def body(x_ref, w_ref, b_ref, o_ref):
        o_ref[...] = jnp.dot(x_ref[...], w_ref[...],
                             preferred_element_type=jnp.float32) + b_ref[...]
    return pl.pallas_call(
        body,
        grid=(N // _BR,),
        in_specs=[
            pl.BlockSpec((_BR, F), lambda i: (i, 0)),
            pl.BlockSpec((F, F), lambda i: (0, 0)),
            pl.BlockSpec((1, F), lambda i: (0, 0)),
        ],
        out_specs=pl.BlockSpec((_BR, F), lambda i: (i, 0)),
        out_shape=jax.ShapeDtypeStruct((N, F), jnp.float32),
    )(x, W, b)


def _tc_leaky_linear(p, W, b):
    def body(p_ref, w_ref, b_ref, o_ref):
        h = p_ref[...]
        h = jnp.where(h > 0, h, 0.2 * h)
        o_ref[...] = jnp.dot(h, w_ref[...],
                             preferred_element_type=jnp.float32) + b_ref[...]
    return pl.pallas_call(
        body,
        grid=(N // _BR,),
        in_specs=[
            pl.BlockSpec((_BR, F), lambda i: (i, 0)),
            pl.BlockSpec((F, F), lambda i: (0, 0)),
            pl.BlockSpec((1, F), lambda i: (0, 0)),
        ],
        out_specs=pl.BlockSpec((_BR, F), lambda i: (i, 0)),
        out_shape=jax.ShapeDtypeStruct((N, F), jnp.float32),
    )(p, W, b)


def _tc_l2norm(p):
    def body(p_ref, o_ref):
        y = p_ref[...]
        nrm = jnp.sqrt(jnp.sum(y * y, axis=1, keepdims=True))
        o_ref[...] = y / jnp.maximum(nrm, 1e-12)
    return pl.pallas_call(
        body,
        grid=(N // _BR,),
        in_specs=[pl.BlockSpec((_BR, F), lambda i: (i, 0))],
        out_specs=pl.BlockSpec((_BR, F), lambda i: (i, 0)),
        out_shape=jax.ShapeDtypeStruct((N, F), jnp.float32),
    )(p)


def kernel(x, W1, b1, W2, b2, a_att, r_att, aux, edges):
    src = edges[:, 0]
    dst = edges[:, 1]
    valst = aux.T.reshape(DA * E)
    att_log = jnp.concatenate([a_att, r_att])[:, None] * jnp.ones(
        (1, F), jnp.float32)

    parts = _sc_rowsums(dst, src, valst)
    rinv, attw = _tc_reduce_inv(parts.reshape(NW, NCH * NP), att_log)
    wa, wr = _sc_weights(rinv.reshape(NCH * NP), attw.reshape(NCH * F),
                         dst, src, valst)

    sup1 = _tc_linear(x, W1, b1.reshape(1, F))
    p1 = _sc_spmm(sup1, src, dst, wa)
    sup2 = _tc_leaky_linear(p1[:N], W2, b2.reshape(1, F))
    p2 = _sc_spmm(sup2, dst, src, wr)
    return _tc_l2norm(p2[:N])


# speedup vs baseline: 20.7963x; 1.0305x over previous
import functools

import jax
import jax.numpy as jnp
from jax import lax
from jax.experimental import pallas as pl
from jax.experimental.pallas import tpu as pltpu
from jax.experimental.pallas import tpu_sc as plsc

N = 10000
NP = 10240
NPH = NP // 2
E = 320000
F = 128
DA = 4
NCH = 2 * DA
NC, NS, L = 2, 16, 16
NW = NC * NS
EPW = E // NW
EPS = E // NS
RPT = NPH // NS

_mesh = plsc.VectorSubcoreMesh(
    core_axis_name="c", subcore_axis_name="s", num_cores=NC, num_subcores=NS)

_sc_params = pltpu.CompilerParams(needs_layout_passes=False)


def _wid():
    return lax.axis_index("s") * NC + lax.axis_index("c")


_CA = 2000


@functools.partial(
    pl.kernel,
    out_type=jax.ShapeDtypeStruct((NW * NCH * NP,), jnp.float32),
    mesh=_mesh,
    compiler_params=_sc_params,
    scratch_types=[
        pltpu.VMEM((NCH * NP,), jnp.float32),
        pltpu.VMEM((_CA,), jnp.int32),
        pltpu.VMEM((_CA,), jnp.int32),
        pltpu.VMEM((DA * _CA,), jnp.float32),
    ],
)
def _sc_rowsums(dst_h, src_h, valst_h, parts_h, acc, dbuf, sbuf, vbuf):
    wid = _wid()
    zf = jnp.zeros((L,), jnp.float32)

    def _z(j, _):
        acc[pl.ds(j * L, L)] = zf
        return _
    lax.fori_loop(0, NCH * NP // L, _z, None)

    def _chunk(g, _):
        base = wid * EPW + g * _CA
        pltpu.sync_copy(dst_h.at[pl.ds(base, _CA)], dbuf)
        pltpu.sync_copy(src_h.at[pl.ds(base, _CA)], sbuf)
        for i in range(DA):
            pltpu.sync_copy(valst_h.at[pl.ds(i * E + base, _CA)],
                            vbuf.at[pl.ds(i * _CA, _CA)])

        def _vec(j, _):
            dv = dbuf[pl.ds(j * L, L)]
            sv = sbuf[pl.ds(j * L, L)]
            for i in range(DA):
                v = vbuf[pl.ds(i * _CA + j * L, L)]
                if i == 0:
                    v = jnp.where(v == 0.0, 0.1, v)
                plsc.addupdate_scatter(acc, [dv + i * NP], v)
                plsc.addupdate_scatter(acc, [sv + (DA + i) * NP], v)
            return _
        lax.fori_loop(0, _CA // L, _vec, None)
        return _
    lax.fori_loop(0, EPW // _CA, _chunk, None)

    pltpu.sync_copy(acc, parts_h.at[pl.ds(wid * (NCH * NP), NCH * NP)])


def _tc_reduce_inv(parts, att_log):
    def body(p_ref, a_ref, rinv_ref, attw_ref):
        s = jnp.sum(p_ref[...], axis=0, keepdims=True)
        rinv_ref[...] = 1.0 / s
        a = a_ref[0:DA, :]
        r = a_ref[DA:NCH, :]
        ma = jnp.max(a, axis=0, keepdims=True)
        mr = jnp.max(r, axis=0, keepdims=True)
        ea = jnp.exp(a - ma)
        er = jnp.exp(r - mr)
        wa = ea / jnp.sum(ea, axis=0, keepdims=True)
        wr = er / jnp.sum(er, axis=0, keepdims=True)
        attw_ref[...] = jnp.concatenate([wa, wr], axis=0)

    return pl.pallas_call(
        body,
        out_shape=[jax.ShapeDtypeStruct((1, NCH * NP), jnp.float32),
                   jax.ShapeDtypeStruct((NCH, F), jnp.float32)],
    )(parts, att_log)


_CB = 2000


@functools.partial(
    pl.kernel,
    out_type=[jax.ShapeDtypeStruct((E,), jnp.float32),
              jax.ShapeDtypeStruct((E,), jnp.float32)],
    mesh=_mesh,
    compiler_params=_sc_params,
    scratch_types=[
        pltpu.VMEM((NCH * NP,), jnp.float32),
        pltpu.VMEM((NCH * F,), jnp.float32),
        pltpu.VMEM((_CB,), jnp.int32),
        pltpu.VMEM((_CB,), jnp.int32),
        pltpu.VMEM((DA * _CB,), jnp.float32),
        pltpu.VMEM((_CB,), jnp.float32),
        pltpu.VMEM((_CB,), jnp.float32),
    ],
)
def _sc_weights(rinv_h, attw_h, dst_h, src_h, valst_h, wa_h, wr_h,
                rinv, attw, dbuf, sbuf, vbuf, wabuf, wrbuf):
    wid = _wid()
    pltpu.sync_copy(rinv_h, rinv)
    pltpu.sync_copy(attw_h, attw)

    def _chunk(g, _):
        base = wid * EPW + g * _CB
        pltpu.sync_copy(dst_h.at[pl.ds(base, _CB)], dbuf)
        pltpu.sync_copy(src_h.at[pl.ds(base, _CB)], sbuf)
        for i in range(DA):
            pltpu.sync_copy(valst_h.at[pl.ds(i * E + base, _CB)],
                            vbuf.at[pl.ds(i * _CB, _CB)])

        def _vec(j, _):
            dv = dbuf[pl.ds(j * L, L)]
            sv = sbuf[pl.ds(j * L, L)]
            wa = jnp.zeros((L,), jnp.float32)
            wr = jnp.zeros((L,), jnp.float32)
            for i in range(DA):
                v = vbuf[pl.ds(i * _CB + j * L, L)]
                if i == 0:
                    v = jnp.where(v == 0.0, 0.1, v)
                ia = plsc.load_gather(rinv, [dv + i * NP])
                ir = plsc.load_gather(rinv, [sv + (DA + i) * NP])
                wa = wa + v * ia * attw[pl.ds(i * F, L)]
                wr = wr + v * ir * attw[pl.ds((DA + i) * F, L)]
            wabuf[pl.ds(j * L, L)] = wa
            wrbuf[pl.ds(j * L, L)] = wr
            return _
        lax.fori_loop(0, _CB // L, _vec, None)
        pltpu.sync_copy(wabuf, wa_h.at[pl.ds(base, _CB)])
        pltpu.sync_copy(wrbuf, wr_h.at[pl.ds(base, _CB)])
        return _
    lax.fori_loop(0, EPW // _CB, _chunk, None)


_CC = 400
_ZR = 64


@functools.partial(
    pl.kernel,
    out_type=jax.ShapeDtypeStruct((NP, F), jnp.float32),
    mesh=_mesh,
    compiler_params=_sc_params,
    scratch_types=[
        pltpu.VMEM((_ZR, F), jnp.float32),
        pltpu.VMEM((_CC, F), jnp.float32),
        pltpu.VMEM((_CC,), jnp.int32),
        pltpu.VMEM((_CC,), jnp.int32),
        pltpu.VMEM((_CC,), jnp.float32),
        pltpu.VMEM_SHARED((NPH + 8, F), jnp.float32),
        pltpu.SemaphoreType.DMA,
    ],
)
def _sc_spmm(sup_h, gidx_h, sidx_h, w_h, out_h,
             zbuf, rows, gbuf, sbuf, wbuf, acc_sh, sem):
    cid = lax.axis_index("c")
    sid = lax.axis_index("s")
    zf = jnp.zeros((L,), jnp.float32)

    def _z(r, _):
        for k in range(F // L):
            zbuf[r, pl.ds(k * L, L)] = zf
        return _
    lax.fori_loop(0, _ZR, _z, None)
    for j in range(RPT // _ZR):
        pltpu.sync_copy(zbuf, acc_sh.at[pl.ds(sid * RPT + j * _ZR, _ZR)])

    @pl.when(sid == 0)
    def _():
        pltpu.sync_copy(zbuf.at[pl.ds(0, 8)], acc_sh.at[pl.ds(NPH, 8)])

    plsc.subcore_barrier()
    lo = cid * NPH
    ebase = sid * EPS

    def _chunk(g, _):
        base = ebase + g * _CC
        pltpu.sync_copy(gidx_h.at[pl.ds(base, _CC)], gbuf)
        pltpu.async_copy(sup_h.at[gbuf], rows, sem).wait()
        pltpu.sync_copy(w_h.at[pl.ds(base, _CC)], wbuf)
        pltpu.sync_copy(sidx_h.at[pl.ds(base, _CC)], sbuf)

        def _local(j, _):
            lv = sbuf[pl.ds(j * L, L)] - lo
            ok = (lv >= 0) & (lv < NPH)
            sbuf[pl.ds(j * L, L)] = jnp.where(ok, lv, NPH)
            return _
        lax.fori_loop(0, _CC // L, _local, None)

        def _scale(q, _):
            for u in range(4):
                r = q * 4 + u
                wv = plsc.load_gather(
                    wbuf, [lax.broadcast_in_dim(r, (L,), ())])
                for k in range(F // L):
                    rows[r, pl.ds(k * L, L)] = rows[r, pl.ds(k * L, L)] * wv
            return _
        lax.fori_loop(0, _CC // 4, _scale, None)
        pltpu.sync_copy(rows, acc_sh.at[sbuf], add=True)
        return _
    lax.fori_loop(0, EPS // _CC, _chunk, None)

    plsc.subcore_barrier()
    pltpu.sync_copy(acc_sh.at[pl.ds(sid * RPT, RPT)],
                    out_h.at[pl.ds(cid * NPH + sid * RPT, RPT)])


_BR = 2000


def _tc_linear(x, W, b):
    def body(x_ref, w_ref, b_ref, o_ref):
        o_ref[...] = jnp.dot(x_ref[...], w_ref[...],
                             preferred_element_type=jnp.float32) + b_ref[...]
    return pl.pallas_call(
        body,
        grid=(N // _BR,),
        in_specs=[
            pl.BlockSpec((_BR, F), lambda i: (i, 0)),
            pl.BlockSpec((F, F), lambda i: (0, 0)),
            pl.BlockSpec((1, F), lambda i: (0, 0)),
        ],
        out_specs=pl.BlockSpec((_BR, F), lambda i: (i, 0)),
        out_shape=jax.ShapeDtypeStruct((N, F), jnp.float32),
    )(x, W, b)


def _tc_leaky_linear(p, W, b):
    def body(p_ref, w_ref, b_ref, o_ref):
        h = p_ref[...]
        h = jnp.where(h > 0, h, 0.2 * h)
        o_ref[...] = jnp.dot(h, w_ref[...],
                             preferred_element_type=jnp.float32) + b_ref[...]
    return pl.pallas_call(
        body,
        grid=(N // _BR,),
        in_specs=[
            pl.BlockSpec((_BR, F), lambda i: (i, 0)),
            pl.BlockSpec((F, F), lambda i: (0, 0)),
            pl.BlockSpec((1, F), lambda i: (0, 0)),
        ],
        out_specs=pl.BlockSpec((_BR, F), lambda i: (i, 0)),
        out_shape=jax.ShapeDtypeStruct((N, F), jnp.float32),
    )(p, W, b)


def _tc_l2norm(p):
    def body(p_ref, o_ref):
        y = p_ref[...]
        nrm = jnp.sqrt(jnp.sum(y * y, axis=1, keepdims=True))
        o_ref[...] = y / jnp.maximum(nrm, 1e-12)
    return pl.pallas_call(
        body,
        grid=(N // _BR,),
        in_specs=[pl.BlockSpec((_BR, F), lambda i: (i, 0))],
        out_specs=pl.BlockSpec((_BR, F), lambda i: (i, 0)),
        out_shape=jax.ShapeDtypeStruct((N, F), jnp.float32),
    )(p)


def kernel(x, W1, b1, W2, b2, a_att, r_att, aux, edges):
    src = edges[:, 0]
    dst = edges[:, 1]
    valst = aux.T.reshape(DA * E)
    att_log = jnp.concatenate([a_att, r_att])[:, None] * jnp.ones(
        (1, F), jnp.float32)

    parts = _sc_rowsums(dst, src, valst)
    rinv, attw = _tc_reduce_inv(parts.reshape(NW, NCH * NP), att_log)
    wa, wr = _sc_weights(rinv.reshape(NCH * NP), attw.reshape(NCH * F),
                         dst, src, valst)

    sup1 = _tc_linear(x, W1, b1.reshape(1, F))
    p1 = _sc_spmm(sup1, src, dst, wa)
    sup2 = _tc_leaky_linear(p1[:N], W2, b2.reshape(1, F))
    p2 = _sc_spmm(sup2, dst, src, wr)
    return _tc_l2norm(p2[:N])


# speedup vs baseline: 21.9634x; 1.0561x over previous
import functools

import jax
import jax.numpy as jnp
from jax import lax
from jax.experimental import pallas as pl
from jax.experimental.pallas import tpu as pltpu
from jax.experimental.pallas import tpu_sc as plsc

N = 10000
NP = 10240
NPH = NP // 2
E = 320000
F = 128
DA = 4
NCH = 2 * DA
NC, NS, L = 2, 16, 16
NW = NC * NS
EPW = E // NW
EPS = E // NS
RPT = NPH // NS

_mesh = plsc.VectorSubcoreMesh(
    core_axis_name="c", subcore_axis_name="s", num_cores=NC, num_subcores=NS)

_sc_params = pltpu.CompilerParams(needs_layout_passes=False)


def _wid():
    return lax.axis_index("s") * NC + lax.axis_index("c")


_CA = 2000


@functools.partial(
    pl.kernel,
    out_type=jax.ShapeDtypeStruct((NW * NCH * NP,), jnp.float32),
    mesh=_mesh,
    compiler_params=_sc_params,
    scratch_types=[
        pltpu.VMEM((NCH * NP,), jnp.float32),
        pltpu.VMEM((_CA,), jnp.int32),
        pltpu.VMEM((_CA,), jnp.int32),
        pltpu.VMEM((DA * _CA,), jnp.float32),
    ],
)
def _sc_rowsums(dst_h, src_h, valst_h, parts_h, acc, dbuf, sbuf, vbuf):
    wid = _wid()
    zf = jnp.zeros((L,), jnp.float32)

    def _z(j, _):
        acc[pl.ds(j * L, L)] = zf
        return _
    lax.fori_loop(0, NCH * NP // L, _z, None)

    def _chunk(g, _):
        base = wid * EPW + g * _CA
        pltpu.sync_copy(dst_h.at[pl.ds(base, _CA)], dbuf)
        pltpu.sync_copy(src_h.at[pl.ds(base, _CA)], sbuf)
        for i in range(DA):
            pltpu.sync_copy(valst_h.at[pl.ds(i * E + base, _CA)],
                            vbuf.at[pl.ds(i * _CA, _CA)])

        def _vec(j, _):
            dv = dbuf[pl.ds(j * L, L)]
            sv = sbuf[pl.ds(j * L, L)]
            for i in range(DA):
                v = vbuf[pl.ds(i * _CA + j * L, L)]
                if i == 0:
                    v = jnp.where(v == 0.0, 0.1, v)
                plsc.addupdate_scatter(acc, [dv + i * NP], v)
                plsc.addupdate_scatter(acc, [sv + (DA + i) * NP], v)
            return _
        lax.fori_loop(0, _CA // L, _vec, None)
        return _
    lax.fori_loop(0, EPW // _CA, _chunk, None)

    pltpu.sync_copy(acc, parts_h.at[pl.ds(wid * (NCH * NP), NCH * NP)])


def _tc_reduce_inv(parts, att_log):
    def body(p_ref, a_ref, rinv_ref, attw_ref):
        s = jnp.sum(p_ref[...], axis=0, keepdims=True)
        rinv_ref[...] = 1.0 / s
        a = a_ref[0:DA, :]
        r = a_ref[DA:NCH, :]
        ma = jnp.max(a, axis=0, keepdims=True)
        mr = jnp.max(r, axis=0, keepdims=True)
        ea = jnp.exp(a - ma)
        er = jnp.exp(r - mr)
        wa = ea / jnp.sum(ea, axis=0, keepdims=True)
        wr = er / jnp.sum(er, axis=0, keepdims=True)
        attw_ref[...] = jnp.concatenate([wa, wr], axis=0)

    return pl.pallas_call(
        body,
        out_shape=[jax.ShapeDtypeStruct((1, NCH * NP), jnp.float32),
                   jax.ShapeDtypeStruct((NCH, F), jnp.float32)],
    )(parts, att_log)


_CB = 2000


@functools.partial(
    pl.kernel,
    out_type=[jax.ShapeDtypeStruct((E,), jnp.float32),
              jax.ShapeDtypeStruct((E,), jnp.float32)],
    mesh=_mesh,
    compiler_params=_sc_params,
    scratch_types=[
        pltpu.VMEM((NCH * NP,), jnp.float32),
        pltpu.VMEM((NCH * F,), jnp.float32),
        pltpu.VMEM((_CB,), jnp.int32),
        pltpu.VMEM((_CB,), jnp.int32),
        pltpu.VMEM((DA * _CB,), jnp.float32),
        pltpu.VMEM((_CB,), jnp.float32),
        pltpu.VMEM((_CB,), jnp.float32),
    ],
)
def _sc_weights(rinv_h, attw_h, dst_h, src_h, valst_h, wa_h, wr_h,
                rinv, attw, dbuf, sbuf, vbuf, wabuf, wrbuf):
    wid = _wid()
    pltpu.sync_copy(rinv_h, rinv)
    pltpu.sync_copy(attw_h, attw)

    def _chunk(g, _):
        base = wid * EPW + g * _CB
        pltpu.sync_copy(dst_h.at[pl.ds(base, _CB)], dbuf)
        pltpu.sync_copy(src_h.at[pl.ds(base, _CB)], sbuf)
        for i in range(DA):
            pltpu.sync_copy(valst_h.at[pl.ds(i * E + base, _CB)],
                            vbuf.at[pl.ds(i * _CB, _CB)])

        def _vec(j, _):
            dv = dbuf[pl.ds(j * L, L)]
            sv = sbuf[pl.ds(j * L, L)]
            wa = jnp.zeros((L,), jnp.float32)
            wr = jnp.zeros((L,), jnp.float32)
            for i in range(DA):
                v = vbuf[pl.ds(i * _CB + j * L, L)]
                if i == 0:
                    v = jnp.where(v == 0.0, 0.1, v)
                ia = plsc.load_gather(rinv, [dv + i * NP])
                ir = plsc.load_gather(rinv, [sv + (DA + i) * NP])
                wa = wa + v * ia * attw[pl.ds(i * F, L)]
                wr = wr + v * ir * attw[pl.ds((DA + i) * F, L)]
            wabuf[pl.ds(j * L, L)] = wa
            wrbuf[pl.ds(j * L, L)] = wr
            return _
        lax.fori_loop(0, _CB // L, _vec, None)
        pltpu.sync_copy(wabuf, wa_h.at[pl.ds(base, _CB)])
        pltpu.sync_copy(wrbuf, wr_h.at[pl.ds(base, _CB)])
        return _
    lax.fori_loop(0, EPW // _CB, _chunk, None)


_CC = 400
_NCH_C = EPS // _CC
_ZR = 32


@functools.partial(
    pl.kernel,
    out_type=jax.ShapeDtypeStruct((NP, F), jnp.float32),
    mesh=_mesh,
    compiler_params=_sc_params,
    scratch_types=[
        pltpu.VMEM((_ZR, F), jnp.float32),
        pltpu.VMEM((_CC, F), jnp.float32),
        pltpu.VMEM((3 * _CC,), jnp.int32),
        pltpu.VMEM((_CC,), jnp.int32),
        pltpu.VMEM_SHARED((NPH + 8, F), jnp.float32),
        pltpu.SemaphoreType.DMA,
    ],
)
def _sc_spmm(sup_h, packed_h, out_h,
             zbuf, rows, pbuf, sbuf, acc_sh, sem):
    cid = lax.axis_index("c")
    sid = lax.axis_index("s")
    zf = jnp.zeros((L,), jnp.float32)

    def _z(r, _):
        for k in range(F // L):
            zbuf[r, pl.ds(k * L, L)] = zf
        return _
    lax.fori_loop(0, _ZR, _z, None)
    for j in range(RPT // _ZR):
        pltpu.sync_copy(zbuf, acc_sh.at[pl.ds(sid * RPT + j * _ZR, _ZR)])

    @pl.when(sid == 0)
    def _():
        pltpu.sync_copy(zbuf.at[pl.ds(0, 8)], acc_sh.at[pl.ds(NPH, 8)])

    plsc.subcore_barrier()
    lo = cid * NPH

    def _chunk(g, _):
        pltpu.sync_copy(
            packed_h.at[pl.ds((sid * _NCH_C + g) * (3 * _CC), 3 * _CC)],
            pbuf)
        pltpu.async_copy(sup_h.at[pbuf.at[pl.ds(0, _CC)]], rows, sem).wait()

        def _local(j, _):
            lv = pbuf[pl.ds(_CC + j * L, L)] - lo
            ok = (lv >= 0) & (lv < NPH)
            sbuf[pl.ds(j * L, L)] = jnp.where(ok, lv, NPH)
            return _
        lax.fori_loop(0, _CC // L, _local, None)

        def _scale(q, _):
            for u in range(4):
                r = q * 4 + u
                wv = plsc.bitcast(
                    plsc.load_gather(
                        pbuf,
                        [lax.broadcast_in_dim(2 * _CC + r, (L,), ())]),
                    jnp.float32)
                for k in range(F // L):
                    rows[r, pl.ds(k * L, L)] = rows[r, pl.ds(k * L, L)] * wv
            return _
        lax.fori_loop(0, _CC // 4, _scale, None)
        pltpu.sync_copy(rows, acc_sh.at[sbuf], add=True)
        return _
    lax.fori_loop(0, _NCH_C, _chunk, None)

    plsc.subcore_barrier()
    pltpu.sync_copy(acc_sh.at[pl.ds(sid * RPT, RPT)],
                    out_h.at[pl.ds(cid * NPH + sid * RPT, RPT)])


_BR = 2000


def _tc_linear(x, W, b):
    def body(x_ref, w_ref, b_ref, o_ref):
        o_ref[...] = jnp.dot(x_ref[...], w_ref[...],
                             preferred_element_type=jnp.float32) + b_ref[...]
    return pl.pallas_call(
        body,
        grid=(N // _BR,),
        in_specs=[
            pl.BlockSpec((_BR, F), lambda i: (i, 0)),
            pl.BlockSpec((F, F), lambda i: (0, 0)),
            pl.BlockSpec((1, F), lambda i: (0, 0)),
        ],
        out_specs=pl.BlockSpec((_BR, F), lambda i: (i, 0)),
        out_shape=jax.ShapeDtypeStruct((N, F), jnp.float32),
    )(x, W, b)


def _tc_leaky_linear(p, W, b):
    def body(p_ref, w_ref, b_ref, o_ref):
        h = p_ref[...]
        h = jnp.where(h > 0, h, 0.2 * h)
        o_ref[...] = jnp.dot(h, w_ref[...],
                             preferred_element_type=jnp.float32) + b_ref[...]
    return pl.pallas_call(
        body,
        grid=(N // _BR,),
        in_specs=[
            pl.BlockSpec((_BR, F), lambda i: (i, 0)),
            pl.BlockSpec((F, F), lambda i: (0, 0)),
            pl.BlockSpec((1, F), lambda i: (0, 0)),
        ],
        out_specs=pl.BlockSpec((_BR, F), lambda i: (i, 0)),
        out_shape=jax.ShapeDtypeStruct((N, F), jnp.float32),
    )(p, W, b)


def _tc_l2norm(p):
    def body(p_ref, o_ref):
        y = p_ref[...]
        nrm = jnp.sqrt(jnp.sum(y * y, axis=1, keepdims=True))
        o_ref[...] = y / jnp.maximum(nrm, 1e-12)
    return pl.pallas_call(
        body,
        grid=(N // _BR,),
        in_specs=[pl.BlockSpec((_BR, F), lambda i: (i, 0))],
        out_specs=pl.BlockSpec((_BR, F), lambda i: (i, 0)),
        out_shape=jax.ShapeDtypeStruct((N, F), jnp.float32),
    )(p)


def kernel(x, W1, b1, W2, b2, a_att, r_att, aux, edges):
    src = edges[:, 0]
    dst = edges[:, 1]
    valst = aux.T.reshape(DA * E)
    att_log = jnp.concatenate([a_att, r_att])[:, None] * jnp.ones(
        (1, F), jnp.float32)

    parts = _sc_rowsums(dst, src, valst)
    rinv, attw = _tc_reduce_inv(parts.reshape(NW, NCH * NP), att_log)
    wa, wr = _sc_weights(rinv.reshape(NCH * NP), attw.reshape(NCH * F),
                         dst, src, valst)

    def _pack(g, s, w):
        gr = g.reshape(NS, _NCH_C, _CC)
        sr = s.reshape(NS, _NCH_C, _CC)
        wb = lax.bitcast_convert_type(w, jnp.int32).reshape(NS, _NCH_C, _CC)
        return jnp.stack([gr, sr, wb], axis=2).reshape(-1)

    sup1 = _tc_linear(x, W1, b1.reshape(1, F))
    p1 = _sc_spmm(sup1, _pack(src, dst, wa))
    sup2 = _tc_leaky_linear(p1[:N], W2, b2.reshape(1, F))
    p2 = _sc_spmm(sup2, _pack(dst, src, wr))
    return _tc_l2norm(p2[:N])


# speedup vs baseline: 22.0802x; 1.0053x over previous
import functools

import jax
import jax.numpy as jnp
from jax import lax
from jax.experimental import pallas as pl
from jax.experimental.pallas import tpu as pltpu
from jax.experimental.pallas import tpu_sc as plsc

N = 10000
NP = 10240
NPH = NP // 2
E = 320000
F = 128
DA = 4
NCH = 2 * DA
NC, NS, L = 2, 16, 16
NW = NC * NS
EPW = E // NW
EPS = E // NS
RPT = NPH // NS

_mesh = plsc.VectorSubcoreMesh(
    core_axis_name="c", subcore_axis_name="s", num_cores=NC, num_subcores=NS)

_sc_params = pltpu.CompilerParams(needs_layout_passes=False)


def _wid():
    return lax.axis_index("s") * NC + lax.axis_index("c")


_CA = 2000


@functools.partial(
    pl.kernel,
    out_type=jax.ShapeDtypeStruct((NW * NCH * NP,), jnp.float32),
    mesh=_mesh,
    compiler_params=_sc_params,
    scratch_types=[
        pltpu.VMEM((NCH * NP,), jnp.float32),
        pltpu.VMEM((_CA,), jnp.int32),
        pltpu.VMEM((_CA,), jnp.int32),
        pltpu.VMEM((DA * _CA,), jnp.float32),
    ],
)
def _sc_rowsums(dst_h, src_h, valst_h, parts_h, acc, dbuf, sbuf, vbuf):
    wid = _wid()
    zf = jnp.zeros((L,), jnp.float32)

    def _z(j, _):
        acc[pl.ds(j * L, L)] = zf
        return _
    lax.fori_loop(0, NCH * NP // L, _z, None)

    def _chunk(g, _):
        base = wid * EPW + g * _CA
        pltpu.sync_copy(dst_h.at[pl.ds(base, _CA)], dbuf)
        pltpu.sync_copy(src_h.at[pl.ds(base, _CA)], sbuf)
        for i in range(DA):
            pltpu.sync_copy(valst_h.at[pl.ds(i * E + base, _CA)],
                            vbuf.at[pl.ds(i * _CA, _CA)])

        def _vec(j, _):
            dv = dbuf[pl.ds(j * L, L)]
            sv = sbuf[pl.ds(j * L, L)]
            for i in range(DA):
                v = vbuf[pl.ds(i * _CA + j * L, L)]
                if i == 0:
                    v = jnp.where(v == 0.0, 0.1, v)
                plsc.addupdate_scatter(acc, [dv + i * NP], v)
                plsc.addupdate_scatter(acc, [sv + (DA + i) * NP], v)
            return _
        lax.fori_loop(0, _CA // L, _vec, None)
        return _
    lax.fori_loop(0, EPW // _CA, _chunk, None)

    pltpu.sync_copy(acc, parts_h.at[pl.ds(wid * (NCH * NP), NCH * NP)])


def _tc_reduce_inv(parts, att_log):
    def body(p_ref, a_ref, rinv_ref, attw_ref):
        s = jnp.sum(p_ref[...], axis=0, keepdims=True)
        rinv_ref[...] = 1.0 / s
        a = a_ref[0:DA, :]
        r = a_ref[DA:NCH, :]
        ma = jnp.max(a, axis=0, keepdims=True)
        mr = jnp.max(r, axis=0, keepdims=True)
        ea = jnp.exp(a - ma)
        er = jnp.exp(r - mr)
        wa = ea / jnp.sum(ea, axis=0, keepdims=True)
        wr = er / jnp.sum(er, axis=0, keepdims=True)
        attw_ref[...] = jnp.concatenate([wa, wr], axis=0)

    return pl.pallas_call(
        body,
        out_shape=[jax.ShapeDtypeStruct((1, NCH * NP), jnp.float32),
                   jax.ShapeDtypeStruct((NCH, F), jnp.float32)],
    )(parts, att_log)


_CB = 2000


@functools.partial(
    pl.kernel,
    out_type=[jax.ShapeDtypeStruct((E,), jnp.float32),
              jax.ShapeDtypeStruct((E,), jnp.float32)],
    mesh=_mesh,
    compiler_params=_sc_params,
    scratch_types=[
        pltpu.VMEM((NCH * NP,), jnp.float32),
        pltpu.VMEM((NCH * F,), jnp.float32),
        pltpu.VMEM((_CB,), jnp.int32),
        pltpu.VMEM((_CB,), jnp.int32),
        pltpu.VMEM((DA * _CB,), jnp.float32),
        pltpu.VMEM((_CB,), jnp.float32),
        pltpu.VMEM((_CB,), jnp.float32),
    ],
)
def _sc_weights(rinv_h, attw_h, dst_h, src_h, valst_h, wa_h, wr_h,
                rinv, attw, dbuf, sbuf, vbuf, wabuf, wrbuf):
    wid = _wid()
    pltpu.sync_copy(rinv_h, rinv)
    pltpu.sync_copy(attw_h, attw)

    def _chunk(g, _):
        base = wid * EPW + g * _CB
        pltpu.sync_copy(dst_h.at[pl.ds(base, _CB)], dbuf)
        pltpu.sync_copy(src_h.at[pl.ds(base, _CB)], sbuf)
        for i in range(DA):
            pltpu.sync_copy(valst_h.at[pl.ds(i * E + base, _CB)],
                            vbuf.at[pl.ds(i * _CB, _CB)])

        def _vec(j, _):
            dv = dbuf[pl.ds(j * L, L)]
            sv = sbuf[pl.ds(j * L, L)]
            wa = jnp.zeros((L,), jnp.float32)
            wr = jnp.zeros((L,), jnp.float32)
            for i in range(DA):
                v = vbuf[pl.ds(i * _CB + j * L, L)]
                if i == 0:
                    v = jnp.where(v == 0.0, 0.1, v)
                ia = plsc.load_gather(rinv, [dv + i * NP])
                ir = plsc.load_gather(rinv, [sv + (DA + i) * NP])
                wa = wa + v * ia * attw[pl.ds(i * F, L)]
                wr = wr + v * ir * attw[pl.ds((DA + i) * F, L)]
            wabuf[pl.ds(j * L, L)] = wa
            wrbuf[pl.ds(j * L, L)] = wr
            return _
        lax.fori_loop(0, _CB // L, _vec, None)
        pltpu.sync_copy(wabuf, wa_h.at[pl.ds(base, _CB)])
        pltpu.sync_copy(wrbuf, wr_h.at[pl.ds(base, _CB)])
        return _
    lax.fori_loop(0, EPW // _CB, _chunk, None)


_CC = 400
_NCH_C = EPS // _CC
_ZR = 32


@functools.partial(
    pl.kernel,
    out_type=jax.ShapeDtypeStruct((NP, F), jnp.float32),
    mesh=_mesh,
    compiler_params=_sc_params,
    scratch_types=[
        pltpu.VMEM((_ZR, F), jnp.float32),
        pltpu.VMEM((_CC, F), jnp.float32),
        pltpu.VMEM((3 * _CC,), jnp.int32),
        pltpu.VMEM((_CC,), jnp.int32),
        pltpu.VMEM_SHARED((NPH + 8, F), jnp.float32),
        pltpu.SemaphoreType.DMA,
    ],
)
def _sc_spmm(sup_h, packed_h, out_h,
             zbuf, rows, pbuf, sbuf, acc_sh, sem):
    cid = lax.axis_index("c")
    sid = lax.axis_index("s")
    zf = jnp.zeros((L,), jnp.float32)

    def _z(r, _):
        for k in range(F // L):
            zbuf[r, pl.ds(k * L, L)] = zf
        return _
    lax.fori_loop(0, _ZR, _z, None)
    for j in range(RPT // _ZR):
        pltpu.sync_copy(zbuf, acc_sh.at[pl.ds(sid * RPT + j * _ZR, _ZR)])

    @pl.when(sid == 0)
    def _():
        pltpu.sync_copy(zbuf.at[pl.ds(0, 8)], acc_sh.at[pl.ds(NPH, 8)])

    plsc.subcore_barrier()
    lo = cid * NPH

    def _chunk(g, _):
        pltpu.sync_copy(
            packed_h.at[pl.ds((sid * _NCH_C + g) * (3 * _CC), 3 * _CC)],
            pbuf)
        pltpu.async_copy(sup_h.at[pbuf.at[pl.ds(0, _CC)]], rows, sem).wait()

        def _local(j, _):
            lv = pbuf[pl.ds(_CC + j * L, L)] - lo
            ok = (lv >= 0) & (lv < NPH)
            sbuf[pl.ds(j * L, L)] = jnp.where(ok, lv, NPH)
            return _
        lax.fori_loop(0, _CC // L, _local, None)

        def _scale(q, _):
            for u in range(8):
                r = q * 8 + u
                wv = plsc.bitcast(
                    plsc.load_gather(
                        pbuf,
                        [lax.broadcast_in_dim(2 * _CC + r, (L,), ())]),
                    jnp.float32)
                for k in range(F // L):
                    rows[r, pl.ds(k * L, L)] = rows[r, pl.ds(k * L, L)] * wv
            return _
        lax.fori_loop(0, _CC // 8, _scale, None)
        pltpu.sync_copy(rows, acc_sh.at[sbuf], add=True)
        return _
    lax.fori_loop(0, _NCH_C, _chunk, None)

    plsc.subcore_barrier()
    pltpu.sync_copy(acc_sh.at[pl.ds(sid * RPT, RPT)],
                    out_h.at[pl.ds(cid * NPH + sid * RPT, RPT)])


_BR = 2000


def _tc_linear(x, W, b):
    def body(x_ref, w_ref, b_ref, o_ref):
        o_ref[...] = jnp.dot(x_ref[...], w_ref[...],
                             preferred_element_type=jnp.float32) + b_ref[...]
    return pl.pallas_call(
        body,
        grid=(N // _BR,),
        in_specs=[
            pl.BlockSpec((_BR, F), lambda i: (i, 0)),
            pl.BlockSpec((F, F), lambda i: (0, 0)),
            pl.BlockSpec((1, F), lambda i: (0, 0)),
        ],
        out_specs=pl.BlockSpec((_BR, F), lambda i: (i, 0)),
        out_shape=jax.ShapeDtypeStruct((N, F), jnp.float32),
    )(x, W, b)


def _tc_leaky_linear(p, W, b):
    def body(p_ref, w_ref, b_ref, o_ref):
        h = p_ref[...]
        h = jnp.where(h > 0, h, 0.2 * h)
        o_ref[...] = jnp.dot(h, w_ref[...],
                             preferred_element_type=jnp.float32) + b_ref[...]
    return pl.pallas_call(
        body,
        grid=(N // _BR,),
        in_specs=[
            pl.BlockSpec((_BR, F), lambda i: (i, 0)),
            pl.BlockSpec((F, F), lambda i: (0, 0)),
            pl.BlockSpec((1, F), lambda i: (0, 0)),
        ],
        out_specs=pl.BlockSpec((_BR, F), lambda i: (i, 0)),
        out_shape=jax.ShapeDtypeStruct((N, F), jnp.float32),
    )(p, W, b)


def _tc_l2norm(p):
    def body(p_ref, o_ref):
        y = p_ref[...]
        nrm = jnp.sqrt(jnp.sum(y * y, axis=1, keepdims=True))
        o_ref[...] = y / jnp.maximum(nrm, 1e-12)
    return pl.pallas_call(
        body,
        grid=(N // _BR,),
        in_specs=[pl.BlockSpec((_BR, F), lambda i: (i, 0))],
        out_specs=pl.BlockSpec((_BR, F), lambda i: (i, 0)),
        out_shape=jax.ShapeDtypeStruct((N, F), jnp.float32),
    )(p)


def kernel(x, W1, b1, W2, b2, a_att, r_att, aux, edges):
    src = edges[:, 0]
    dst = edges[:, 1]
    valst = aux.T.reshape(DA * E)
    att_log = jnp.concatenate([a_att, r_att])[:, None] * jnp.ones(
        (1, F), jnp.float32)

    parts = _sc_rowsums(dst, src, valst)
    rinv, attw = _tc_reduce_inv(parts.reshape(NW, NCH * NP), att_log)
    wa, wr = _sc_weights(rinv.reshape(NCH * NP), attw.reshape(NCH * F),
                         dst, src, valst)

    def _pack(g, s, w):
        gr = g.reshape(NS, _NCH_C, _CC)
        sr = s.reshape(NS, _NCH_C, _CC)
        wb = lax.bitcast_convert_type(w, jnp.int32).reshape(NS, _NCH_C, _CC)
        return jnp.stack([gr, sr, wb], axis=2).reshape(-1)

    sup1 = _tc_linear(x, W1, b1.reshape(1, F))
    p1 = _sc_spmm(sup1, _pack(src, dst, wa))
    sup2 = _tc_leaky_linear(p1[:N], W2, b2.reshape(1, F))
    p2 = _sc_spmm(sup2, _pack(dst, src, wr))
    return _tc_l2norm(p2[:N])
